# baseline (speedup 1.0000x reference)
import jax
import jax.numpy as jnp
from jax.experimental import pallas as pl
from jax.experimental.pallas import tpu as pltpu

B = 2
DIM = 256
NH = 8
HD = 32
AHD = 256
TOPK = 4
H = 64
W = 64
HUP = 32
WUP = 32
N = H * W
NUP = HUP * WUP
NFK = NH * 4 * TOPK
SCALE = HD ** -0.5

_F32 = jnp.float32
_BF16 = jnp.bfloat16


def _dot(a, b, lc, rc):
    return jax.lax.dot_general(a, b, (((lc,), (rc,)), ((), ())),
                               preferred_element_type=_F32)


def _pe_compute(kvu, pw_ref, peb_ref):
    vparts = [kvu[:, 2 * HD * h + HD:2 * HD * h + 2 * HD].reshape(HUP, WUP, HD)
              for h in range(NH)]
    arr = jnp.concatenate(vparts, axis=2)
    padded = jnp.pad(arr, ((3, 5), (3, 5), (0, 0)))
    wsh = [padded[:, kw:kw + WUP, :] for kw in range(7)]
    acc = peb_ref[0][None, None, :] * jnp.ones((HUP, WUP, DIM), _F32)
    for kh in range(7):
        for kw in range(7):
            acc = acc + wsh[kw][kh:kh + HUP] * pw_ref[kh * 7 + kw, :][None, None, :]
    prev = jnp.concatenate([acc[0:1], acc[:-1]], axis=0)
    nxt = jnp.concatenate([acc[1:], acc[HUP - 1:HUP]], axis=0)
    r = jnp.stack([0.75 * acc + 0.25 * prev, 0.75 * acc + 0.25 * nxt], axis=1)
    r = r.reshape(H, WUP, DIM)
    prev = jnp.concatenate([r[:, 0:1], r[:, :-1]], axis=1)
    nxt = jnp.concatenate([r[:, 1:], r[:, WUP - 1:WUP]], axis=1)
    up = jnp.stack([0.75 * r + 0.25 * prev, 0.75 * r + 0.25 * nxt], axis=2)
    return up.reshape(N, DIM)


def _routing(qs, kvu, gm):
    css = []
    for h in range(NH):
        qs_h = qs[:, HD * h:HD * h + HD]
        k = kvu[:, 2 * HD * h:2 * HD * h + HD]
        css.append(_dot(qs_h, k, 1, 1))
    vals = jnp.concatenate(css, axis=0) * (1.0 / N) + gm
    lane = jax.lax.broadcasted_iota(jnp.int32, (NH, NUP), 1)
    picks = []
    for _ in range(TOPK):
        m = jnp.max(vals, axis=1, keepdims=True)
        idx = jnp.min(jnp.where(vals == m, lane, jnp.int32(1 << 30)),
                      axis=1, keepdims=True)
        picks.append(idx)
        vals = jnp.where(lane == idx, -jnp.inf, vals)
    pk = jnp.concatenate(picks, axis=1)
    base = (pk // WUP) * 2 * W + (pk % WUP) * 2
    return jnp.concatenate(
        [base, base + 1, base + W, base + W + 1], axis=1)


def _main_kernel(x_ref, u_ref, qw_ref, kvw_ref, qb_ref, kvb_ref, pw_ref,
                 peb_ref, gum_ref, q_out, fkv_out, kvu_out, qs_out, co_ref,
                 pe_out, idx_out):
    nt = pl.program_id(1)
    xt = x_ref[0]
    qt = (_dot(xt, qw_ref[...], 0, 1) + qb_ref[0][None, :]) * SCALE
    q_out[0] = qt
    fkv_out[0] = _dot(xt, kvw_ref[...], 0, 1) + kvb_ref[0][None, :]
    qs = jnp.sum(qt, axis=0, keepdims=True)

    @pl.when(nt == 0)
    def _():
        kvu = _dot(u_ref[0], kvw_ref[...], 0, 1) + kvb_ref[0][None, :]
        kvu_out[0] = kvu
        qs_out[0] = qs
        pe_out[0] = _pe_compute(kvu, pw_ref, peb_ref)

    @pl.when(nt != 0)
    def _():
        qs_out[0] += qs

    kvu = kvu_out[0]
    qb16 = qt.astype(_BF16)
    outs = []
    for h in range(NH):
        qh = qb16[:, HD * h:HD * h + HD]
        k = kvu[:, 2 * HD * h:2 * HD * h + HD].astype(_BF16)
        v = kvu[:, 2 * HD * h + HD:2 * HD * h + 2 * HD].astype(_BF16)
        p = jnp.exp(_dot(qh, k, 1, 1))
        s = jnp.sum(p, axis=1, keepdims=True)
        outs.append(_dot(p.astype(_BF16), v, 1, 0) / s)
    co_ref[0] = jnp.concatenate(outs, axis=1)

    @pl.when(nt == N // 1024 - 1)
    def _():
        idx_out[0] = _routing(qs_out[0], kvu_out[0], gum_ref[0])


def _fine_kernel(iref, q_ref, co_ref, pe_ref, *rest):
    rows = rest[:NFK]
    obd_ref, bdc_ref, bdr_ref, gbt_ref, pw_ref, pb_ref, out_ref = rest[NFK:]
    q = q_ref[0]
    co = co_ref[0]
    tks, tvs = [], []
    for h in range(NH):
        blk = jnp.concatenate([rows[16 * h + s][0, 0, :, :] for s in range(16)],
                              axis=0)
        blk = blk[:, 2 * HD * h:2 * HD * h + 2 * HD]
        tks.append(jnp.pad(blk[:, :HD], ((0, 0), (HD * h, AHD - HD * h - HD))))
        tvs.append(jnp.pad(blk[:, HD:], ((0, 0), (HD * h, AHD - HD * h - HD))))
    tkbd = jnp.concatenate(tks, axis=0)
    tvbd = jnp.concatenate(tvs, axis=0)
    p = jnp.exp(_dot(q, tkbd, 1, 1))
    numer = _dot(p, tvbd, 1, 0)
    denom = _dot(p, obd_ref[...], 1, 0)
    ro = numer / denom
    g = jax.nn.sigmoid(_dot(co, bdc_ref[...], 1, 0)
                       + _dot(ro, bdr_ref[...], 1, 0) + gbt_ref[0][None, :])
    xout = g * ro + (1.0 - g) * co + pe_ref[0]
    out_ref[0] = _dot(pw_ref[...], xout, 1, 1) + pb_ref[0][:, None]


def kernel(x, upper_feat, q_w, q_b, kv_w, kv_b, proj_w, proj_b, pe_w, pe_b,
           gate_w, gate_b):
    xcf = x.reshape(B, DIM, N)
    ucf = upper_feat.reshape(B, DIM, NUP)
    qw2 = q_w.reshape(AHD, DIM)
    kvw2 = kv_w.reshape(2 * AHD, DIM)
    projw2 = proj_w.reshape(DIM, AHD)
    pew2 = jnp.transpose(pe_w.reshape(DIM, 49))
    qb2 = q_b.reshape(1, AHD)
    kvb2 = kv_b.reshape(1, 2 * AHD)
    projb2 = proj_b.reshape(1, DIM)
    peb2 = pe_b.reshape(1, DIM)
    eye8 = jnp.eye(NH, dtype=_F32)
    bdc = jnp.kron(eye8, jnp.transpose(gate_w[:, :HD]))
    bdr = jnp.kron(eye8, jnp.transpose(gate_w[:, HD:]))
    gbt = jnp.tile(gate_b, NH).reshape(1, AHD)
    obd = (jnp.arange(NFK)[:, None] // 16 ==
           jnp.arange(AHD)[None, :] // HD).astype(_F32)
    gum = jax.random.gumbel(jax.random.key(42), (B, NH, NUP), _F32)

    TN = 1024
    NT = N // TN

    q_all, fkv, kvu, qsum, coarse, pe, fidx = pl.pallas_call(
        _main_kernel,
        grid=(B, NT),
        in_specs=[
            pl.BlockSpec((1, DIM, TN), lambda b, t: (b, 0, t)),
            pl.BlockSpec((1, DIM, NUP), lambda b, t: (b, 0, 0)),
            pl.BlockSpec((AHD, DIM), lambda b, t: (0, 0)),
            pl.BlockSpec((2 * AHD, DIM), lambda b, t: (0, 0)),
            pl.BlockSpec((1, AHD), lambda b, t: (0, 0)),
            pl.BlockSpec((1, 2 * AHD), lambda b, t: (0, 0)),
            pl.BlockSpec((49, DIM), lambda b, t: (0, 0)),
            pl.BlockSpec((1, DIM), lambda b, t: (0, 0)),
            pl.BlockSpec((1, NH, NUP), lambda b, t: (b, 0, 0)),
        ],
        out_specs=[
            pl.BlockSpec((1, TN, AHD), lambda b, t: (b, t, 0)),
            pl.BlockSpec((1, TN, 2 * AHD), lambda b, t: (b, t, 0)),
            pl.BlockSpec((1, NUP, 2 * AHD), lambda b, t: (b, 0, 0)),
            pl.BlockSpec((1, 1, AHD), lambda b, t: (b, 0, 0)),
            pl.BlockSpec((1, TN, AHD), lambda b, t: (b, t, 0)),
            pl.BlockSpec((1, N, DIM), lambda b, t: (b, 0, 0)),
            pl.BlockSpec((1, NH, 16), lambda b, t: (b, 0, 0)),
        ],
        out_shape=[
            jax.ShapeDtypeStruct((B, N, AHD), _F32),
            jax.ShapeDtypeStruct((B, N, 2 * AHD), _F32),
            jax.ShapeDtypeStruct((B, NUP, 2 * AHD), _F32),
            jax.ShapeDtypeStruct((B, 1, AHD), _F32),
            jax.ShapeDtypeStruct((B, N, AHD), _F32),
            jax.ShapeDtypeStruct((B, N, DIM), _F32),
            jax.ShapeDtypeStruct((B, NH, 16), jnp.int32),
        ],
    )(xcf, ucf, qw2, kvw2, qb2, kvb2, pew2, peb2, gum)

    fkv4 = fkv.reshape(B, N, 1, 2 * AHD)

    def _row_spec(j):
        return pl.BlockSpec(
            (1, 1, 1, 2 * AHD),
            lambda b, t, iref, j=j: (b, iref[b * NFK + j], 0, 0))

    outcf = pl.pallas_call(
        _fine_kernel,
        grid_spec=pltpu.PrefetchScalarGridSpec(
            num_scalar_prefetch=1,
            grid=(B, NT),
            in_specs=[
                pl.BlockSpec((1, TN, AHD), lambda b, t, iref: (b, t, 0)),
                pl.BlockSpec((1, TN, AHD), lambda b, t, iref: (b, t, 0)),
                pl.BlockSpec((1, TN, DIM), lambda b, t, iref: (b, t, 0)),
            ] + [_row_spec(j) for j in range(NFK)] + [
                pl.BlockSpec((NFK, AHD), lambda b, t, iref: (0, 0)),
                pl.BlockSpec((AHD, AHD), lambda b, t, iref: (0, 0)),
                pl.BlockSpec((AHD, AHD), lambda b, t, iref: (0, 0)),
                pl.BlockSpec((1, AHD), lambda b, t, iref: (0, 0)),
                pl.BlockSpec((DIM, AHD), lambda b, t, iref: (0, 0)),
                pl.BlockSpec((1, DIM), lambda b, t, iref: (0, 0)),
            ],
            out_specs=pl.BlockSpec((1, DIM, TN), lambda b, t, iref: (b, 0, t)),
        ),
        out_shape=jax.ShapeDtypeStruct((B, DIM, N), _F32),
    )(fidx.reshape(-1), q_all, coarse, pe, *([fkv4] * NFK),
      obd, bdc, bdr, gbt, projw2, projb2)

    return outcf.reshape(B, DIM, H, W)


# speedup vs baseline: 1.1859x; 1.1859x over previous
import jax
import jax.numpy as jnp
from jax.experimental import pallas as pl
from jax.experimental.pallas import tpu as pltpu

B = 2
DIM = 256
NH = 8
HD = 32
AHD = 256
TOPK = 4
H = 64
W = 64
HUP = 32
WUP = 32
N = H * W
NUP = HUP * WUP
NFK = NH * 4 * TOPK
SCALE = HD ** -0.5

_F32 = jnp.float32
_BF16 = jnp.bfloat16


def _dot(a, b, lc, rc):
    return jax.lax.dot_general(a, b, (((lc,), (rc,)), ((), ())),
                               preferred_element_type=_F32)


def _pe_compute(kvu, pw_ref, peb_ref):
    vparts = [kvu[:, 2 * HD * h + HD:2 * HD * h + 2 * HD].reshape(HUP, WUP, HD)
              for h in range(NH)]
    arr = jnp.concatenate(vparts, axis=2)
    padded = jnp.pad(arr, ((3, 5), (3, 5), (0, 0)))
    wsh = [padded[:, kw:kw + WUP, :] for kw in range(7)]
    acc = peb_ref[0][None, None, :] * jnp.ones((HUP, WUP, DIM), _F32)
    for kh in range(7):
        for kw in range(7):
            acc = acc + wsh[kw][kh:kh + HUP] * pw_ref[kh * 7 + kw, :][None, None, :]
    prev = jnp.concatenate([acc[0:1], acc[:-1]], axis=0)
    nxt = jnp.concatenate([acc[1:], acc[HUP - 1:HUP]], axis=0)
    r = jnp.stack([0.75 * acc + 0.25 * prev, 0.75 * acc + 0.25 * nxt], axis=1)
    r = r.reshape(H, WUP, DIM)
    prev = jnp.concatenate([r[:, 0:1], r[:, :-1]], axis=1)
    nxt = jnp.concatenate([r[:, 1:], r[:, WUP - 1:WUP]], axis=1)
    up = jnp.stack([0.75 * r + 0.25 * prev, 0.75 * r + 0.25 * nxt], axis=2)
    return up.reshape(N, DIM)


def _main_kernel(x_ref, u_ref, qw_ref, kvw_ref, qb_ref, kvb_ref, pw_ref,
                 peb_ref, q_out, fkv_out, kvu_out, qs_out, co_ref, pe_out):
    nt = pl.program_id(1)
    xt = x_ref[0]
    qt = (_dot(xt, qw_ref[...], 0, 1) + qb_ref[0][None, :]) * SCALE
    q_out[0] = qt
    fkv_out[0] = _dot(xt, kvw_ref[...], 0, 1) + kvb_ref[0][None, :]
    qs = jnp.sum(qt, axis=0, keepdims=True)

    @pl.when(nt == 0)
    def _():
        kvu = _dot(u_ref[0], kvw_ref[...], 0, 1) + kvb_ref[0][None, :]
        kvu_out[0] = kvu
        qs_out[0] = qs
        pe_out[0] = _pe_compute(kvu, pw_ref, peb_ref)

    @pl.when(nt != 0)
    def _():
        qs_out[0] += qs

    kvu = kvu_out[0]
    qb16 = qt.astype(_BF16)
    outs = []
    for h in range(NH):
        qh = qb16[:, HD * h:HD * h + HD]
        k = kvu[:, 2 * HD * h:2 * HD * h + HD].astype(_BF16)
        v = kvu[:, 2 * HD * h + HD:2 * HD * h + 2 * HD].astype(_BF16)
        p = jnp.exp(_dot(qh, k, 1, 1))
        s = jnp.sum(p, axis=1, keepdims=True)
        outs.append(_dot(p.astype(_BF16), v, 1, 0) / s)
    co_ref[0] = jnp.concatenate(outs, axis=1)


def _route_kernel(qs_ref, kvu_ref, gum_ref, fkv_ref, tk_ref, tv_ref):
    kvu = kvu_ref[0]
    css = []
    for h in range(NH):
        qs_h = qs_ref[0][:, HD * h:HD * h + HD]
        k = kvu[:, 2 * HD * h:2 * HD * h + HD]
        css.append(_dot(qs_h, k, 1, 1))
    vals = jnp.concatenate(css, axis=0) * (1.0 / N) + gum_ref[0]
    lane = jax.lax.broadcasted_iota(jnp.int32, (NH, NUP), 1)
    picks = []
    for _ in range(TOPK):
        m = jnp.max(vals, axis=1, keepdims=True)
        idx = jnp.min(jnp.where(vals == m, lane, jnp.int32(1 << 30)),
                      axis=1, keepdims=True)
        picks.append(idx)
        vals = jnp.where(lane == idx, -jnp.inf, vals)
    pk = jnp.concatenate(picks, axis=1)
    bases = (pk // WUP) * 2 * W + (pk % WUP) * 2
    rowio = jax.lax.broadcasted_iota(jnp.int32, (NH, TOPK), 0)
    laneio = jax.lax.broadcasted_iota(jnp.int32, (NH, TOPK), 1)
    for h in range(NH):
        rows = []
        for t in range(TOPK):
            base = jnp.max(jnp.where((rowio == h) & (laneio == t), bases, -1))
            for off in (0, 1, W, W + 1):
                rows.append(fkv_ref[0, pl.ds(base + off, 1), :])
        blk = jnp.concatenate(rows, axis=0)[:, 2 * HD * h:2 * HD * h + 2 * HD]
        padk = jnp.pad(blk[:, :HD], ((0, 0), (HD * h, AHD - HD * h - HD)))
        padv = jnp.pad(blk[:, HD:], ((0, 0), (HD * h, AHD - HD * h - HD)))
        tk_ref[0, 16 * h:16 * h + 16, :] = padk
        tv_ref[0, 16 * h:16 * h + 16, :] = padv


def _fine_kernel(q_ref, co_ref, pe_ref, tk_ref, tv_ref, obd_ref, bdc_ref,
                 bdr_ref, gbt_ref, pw_ref, pb_ref, out_ref):
    q = q_ref[0]
    co = co_ref[0]
    p = jnp.exp(_dot(q, tk_ref[0], 1, 1))
    numer = _dot(p, tv_ref[0], 1, 0)
    denom = _dot(p, obd_ref[...], 1, 0)
    ro = numer / denom
    g = jax.nn.sigmoid(_dot(co, bdc_ref[...], 1, 0)
                       + _dot(ro, bdr_ref[...], 1, 0) + gbt_ref[0][None, :])
    xout = g * ro + (1.0 - g) * co + pe_ref[0]
    out_ref[0] = _dot(pw_ref[...], xout, 1, 1) + pb_ref[0][:, None]


def kernel(x, upper_feat, q_w, q_b, kv_w, kv_b, proj_w, proj_b, pe_w, pe_b,
           gate_w, gate_b):
    xcf = x.reshape(B, DIM, N)
    ucf = upper_feat.reshape(B, DIM, NUP)
    qw2 = q_w.reshape(AHD, DIM)
    kvw2 = kv_w.reshape(2 * AHD, DIM)
    projw2 = proj_w.reshape(DIM, AHD)
    pew2 = jnp.transpose(pe_w.reshape(DIM, 49))
    qb2 = q_b.reshape(1, AHD)
    kvb2 = kv_b.reshape(1, 2 * AHD)
    projb2 = proj_b.reshape(1, DIM)
    peb2 = pe_b.reshape(1, DIM)
    eye8 = jnp.eye(NH, dtype=_F32)
    bdc = jnp.kron(eye8, jnp.transpose(gate_w[:, :HD]))
    bdr = jnp.kron(eye8, jnp.transpose(gate_w[:, HD:]))
    gbt = jnp.tile(gate_b, NH).reshape(1, AHD)
    obd = (jnp.arange(NFK)[:, None] // 16 ==
           jnp.arange(AHD)[None, :] // HD).astype(_F32)
    gum = jax.random.gumbel(jax.random.key(42), (B, NH, NUP), _F32)

    TN = 1024
    NT = N // TN

    q_all, fkv, kvu, qsum, coarse, pe = pl.pallas_call(
        _main_kernel,
        grid=(B, NT),
        in_specs=[
            pl.BlockSpec((1, DIM, TN), lambda b, t: (b, 0, t)),
            pl.BlockSpec((1, DIM, NUP), lambda b, t: (b, 0, 0)),
            pl.BlockSpec((AHD, DIM), lambda b, t: (0, 0)),
            pl.BlockSpec((2 * AHD, DIM), lambda b, t: (0, 0)),
            pl.BlockSpec((1, AHD), lambda b, t: (0, 0)),
            pl.BlockSpec((1, 2 * AHD), lambda b, t: (0, 0)),
            pl.BlockSpec((49, DIM), lambda b, t: (0, 0)),
            pl.BlockSpec((1, DIM), lambda b, t: (0, 0)),
        ],
        out_specs=[
            pl.BlockSpec((1, TN, AHD), lambda b, t: (b, t, 0)),
            pl.BlockSpec((1, TN, 2 * AHD), lambda b, t: (b, t, 0)),
            pl.BlockSpec((1, NUP, 2 * AHD), lambda b, t: (b, 0, 0)),
            pl.BlockSpec((1, 1, AHD), lambda b, t: (b, 0, 0)),
            pl.BlockSpec((1, TN, AHD), lambda b, t: (b, t, 0)),
            pl.BlockSpec((1, N, DIM), lambda b, t: (b, 0, 0)),
        ],
        out_shape=[
            jax.ShapeDtypeStruct((B, N, AHD), _F32),
            jax.ShapeDtypeStruct((B, N, 2 * AHD), _F32),
            jax.ShapeDtypeStruct((B, NUP, 2 * AHD), _F32),
            jax.ShapeDtypeStruct((B, 1, AHD), _F32),
            jax.ShapeDtypeStruct((B, N, AHD), _F32),
            jax.ShapeDtypeStruct((B, N, DIM), _F32),
        ],
    )(xcf, ucf, qw2, kvw2, qb2, kvb2, pew2, peb2)

    tkbd, tvbd = pl.pallas_call(
        _route_kernel,
        grid=(B,),
        in_specs=[
            pl.BlockSpec((1, 1, AHD), lambda b: (b, 0, 0)),
            pl.BlockSpec((1, NUP, 2 * AHD), lambda b: (b, 0, 0)),
            pl.BlockSpec((1, NH, NUP), lambda b: (b, 0, 0)),
            pl.BlockSpec((1, N, 2 * AHD), lambda b: (b, 0, 0)),
        ],
        out_specs=[
            pl.BlockSpec((1, NFK, AHD), lambda b: (b, 0, 0)),
            pl.BlockSpec((1, NFK, AHD), lambda b: (b, 0, 0)),
        ],
        out_shape=[
            jax.ShapeDtypeStruct((B, NFK, AHD), _F32),
            jax.ShapeDtypeStruct((B, NFK, AHD), _F32),
        ],
    )(qsum, kvu, gum, fkv)

    outcf = pl.pallas_call(
        _fine_kernel,
        grid=(B, NT),
        in_specs=[
            pl.BlockSpec((1, TN, AHD), lambda b, t: (b, t, 0)),
            pl.BlockSpec((1, TN, AHD), lambda b, t: (b, t, 0)),
            pl.BlockSpec((1, TN, DIM), lambda b, t: (b, t, 0)),
            pl.BlockSpec((1, NFK, AHD), lambda b, t: (b, 0, 0)),
            pl.BlockSpec((1, NFK, AHD), lambda b, t: (b, 0, 0)),
            pl.BlockSpec((NFK, AHD), lambda b, t: (0, 0)),
            pl.BlockSpec((AHD, AHD), lambda b, t: (0, 0)),
            pl.BlockSpec((AHD, AHD), lambda b, t: (0, 0)),
            pl.BlockSpec((1, AHD), lambda b, t: (0, 0)),
            pl.BlockSpec((DIM, AHD), lambda b, t: (0, 0)),
            pl.BlockSpec((1, DIM), lambda b, t: (0, 0)),
        ],
        out_specs=pl.BlockSpec((1, DIM, TN), lambda b, t: (b, 0, t)),
        out_shape=jax.ShapeDtypeStruct((B, DIM, N), _F32),
    )(q_all, coarse, pe, tkbd, tvbd, obd, bdc, bdr, gbt, projw2, projb2)

    return outcf.reshape(B, DIM, H, W)
